# baseline (speedup 1.0000x reference)
import functools
import math

import jax
import jax.numpy as jnp
from jax.experimental import pallas as pl
from jax.experimental.pallas import tpu as pltpu

_NUM_LANDMARKS = 64
_HALF = 32
_CQ = 64
_KW = _CQ + _HALF
_NEG = float("-inf")


def _ssa_head_kernel(qt_ref, kt_ref, vt_ref, o_ref,
                     band_ref, lmm_ref, kv0_ref, lmk_ref, lmv_ref, sc_ref,
                     *, s, stride):
    h = pl.program_id(0)
    d = qt_ref.shape[-2]
    scale = 1.0 / math.sqrt(d)
    nchunk = s // _CQ

    @pl.when(h == 0)
    def _():
        c = jax.lax.broadcasted_iota(jnp.int32, (_KW, s), 0)
        r = jax.lax.rem(jax.lax.broadcasted_iota(jnp.int32, (_KW, s), 1), _CQ)
        band_ref[...] = jnp.where((c >= r) & (c <= r + _HALF), 0.0, _NEG)
        ll = jax.lax.broadcasted_iota(jnp.int32, (_NUM_LANDMARKS, s), 0)
        rl = jax.lax.broadcasted_iota(jnp.int32, (_NUM_LANDMARKS, s), 1)
        lmm_ref[...] = jnp.where(ll * stride > rl, _NEG, 0.0)

    kt = kt_ref[0, 0]
    vt = vt_ref[0, 0]
    lmk_ref[...] = kt.reshape(d, _NUM_LANDMARKS, stride)[:, :, 0]
    lmv_ref[...] = vt.reshape(d, _NUM_LANDMARKS, stride)[:, :, 0]
    zeros = jnp.zeros((d, _HALF), jnp.float32)
    kv0_ref[0:d, 0:_HALF] = zeros
    kv0_ref[d:, 0:_HALF] = zeros
    kv0_ref[0:d, _HALF:] = kt[:, 0:_CQ]
    kv0_ref[d:, _HALF:] = vt[:, 0:_CQ]

    qt = qt_ref[0, 0] * scale

    lm_sc = jax.lax.dot_general(
        lmk_ref[...], qt, (((0,), (0,)), ((), ())),
        preferred_element_type=jnp.float32,
    ) + lmm_ref[...]
    m2 = jnp.max(lm_sc, axis=0, keepdims=True)
    e2 = jnp.exp(lm_sc - m2)
    w2 = e2 / jnp.sum(e2, axis=0, keepdims=True)
    glob = jax.lax.dot_general(
        w2, lmv_ref[...], (((0,), (1,)), ((), ())),
        preferred_element_type=jnp.float32,
    )

    for c0 in range(nchunk):
        qc = qt[:, c0 * _CQ:(c0 + 1) * _CQ]
        if c0 == 0:
            kc = kv0_ref[0:d, :]
        else:
            kc = kt[:, c0 * _CQ - _HALF:c0 * _CQ + _CQ]
        sc_ref[:, c0 * _CQ:(c0 + 1) * _CQ] = jax.lax.dot_general(
            kc, qc, (((0,), (0,)), ((), ())),
            preferred_element_type=jnp.float32,
        )

    scm = sc_ref[...] + band_ref[...]
    m = jnp.max(scm, axis=0, keepdims=True)
    e = jnp.exp(scm - m)
    sc_ref[...] = e / jnp.sum(e, axis=0, keepdims=True)

    for c0 in range(nchunk):
        wc = sc_ref[:, c0 * _CQ:(c0 + 1) * _CQ]
        if c0 == 0:
            vc = kv0_ref[d:, :]
        else:
            vc = vt[:, c0 * _CQ - _HALF:c0 * _CQ + _CQ]
        loc = jax.lax.dot_general(
            wc, vc, (((0,), (1,)), ((), ())),
            preferred_element_type=jnp.float32,
        )
        o_ref[0, 0, c0 * _CQ:(c0 + 1) * _CQ, :] = (
            loc + glob[c0 * _CQ:(c0 + 1) * _CQ, :]
        ).astype(o_ref.dtype)


@jax.jit
def kernel(query, key, value):
    b, h, s, d = query.shape
    assert b == 1
    stride = s // _NUM_LANDMARKS

    qt = jnp.transpose(query, (0, 1, 3, 2))
    kt = jnp.transpose(key, (0, 1, 3, 2))
    vt = jnp.transpose(value, (0, 1, 3, 2))

    out = pl.pallas_call(
        functools.partial(_ssa_head_kernel, s=s, stride=stride),
        grid=(h,),
        in_specs=[
            pl.BlockSpec((1, 1, d, s), lambda hh: (0, hh, 0, 0)),
            pl.BlockSpec((1, 1, d, s), lambda hh: (0, hh, 0, 0)),
            pl.BlockSpec((1, 1, d, s), lambda hh: (0, hh, 0, 0)),
        ],
        out_specs=pl.BlockSpec((1, 1, s, d), lambda hh: (0, hh, 0, 0)),
        out_shape=jax.ShapeDtypeStruct((b, h, s, d), query.dtype),
        scratch_shapes=[
            pltpu.VMEM((_KW, s), jnp.float32),
            pltpu.VMEM((_NUM_LANDMARKS, s), jnp.float32),
            pltpu.VMEM((2 * d, _KW), jnp.float32),
            pltpu.VMEM((d, _NUM_LANDMARKS), jnp.float32),
            pltpu.VMEM((d, _NUM_LANDMARKS), jnp.float32),
            pltpu.VMEM((_KW, s), jnp.float32),
        ],
    )(qt, kt, vt)
    return out


# speedup vs baseline: 2.8030x; 2.8030x over previous
import functools
import math

import jax
import jax.numpy as jnp
from jax.experimental import pallas as pl
from jax.experimental.pallas import tpu as pltpu

_NUM_LANDMARKS = 64
_HALF = 32
_CQ = 128
_KW = _CQ + _HALF
_NEG = float("-inf")


def _ssa_head_kernel(qt_ref, kt_ref, vt_ref, o_ref,
                     band_ref, lmm_ref, kv0_ref, lmk_ref, lmv_ref, sc_ref,
                     sel_ref, *, s, stride):
    h = pl.program_id(0)
    d = qt_ref.shape[-2]
    scale = 1.0 / math.sqrt(d)
    nchunk = s // _CQ

    @pl.when(h == 0)
    def _():
        c = jax.lax.broadcasted_iota(jnp.int32, (_KW, s), 0)
        r = jax.lax.rem(jax.lax.broadcasted_iota(jnp.int32, (_KW, s), 1), _CQ)
        band_ref[...] = jnp.where((c >= r) & (c <= r + _HALF), 0.0, _NEG)
        ll = jax.lax.broadcasted_iota(jnp.int32, (_NUM_LANDMARKS, s), 0)
        rl = jax.lax.broadcasted_iota(jnp.int32, (_NUM_LANDMARKS, s), 1)
        lmm_ref[...] = jnp.where(ll * stride > rl, _NEG, 0.0)
        rs = jax.lax.broadcasted_iota(jnp.int32, (s, _NUM_LANDMARKS), 0)
        ls = jax.lax.broadcasted_iota(jnp.int32, (s, _NUM_LANDMARKS), 1)
        sel_ref[...] = (rs == ls * stride).astype(jnp.float32)

    kt = kt_ref[0, 0]
    vt = vt_ref[0, 0]
    lmk_ref[...] = jax.lax.dot_general(
        kt, sel_ref[...], (((1,), (0,)), ((), ())),
        preferred_element_type=jnp.float32)
    lmv_ref[...] = jax.lax.dot_general(
        vt, sel_ref[...], (((1,), (0,)), ((), ())),
        preferred_element_type=jnp.float32)
    zeros = jnp.zeros((d, _HALF), jnp.float32)
    kv0_ref[0:d, 0:_HALF] = zeros
    kv0_ref[d:, 0:_HALF] = zeros
    kv0_ref[0:d, _HALF:] = kt[:, 0:_CQ]
    kv0_ref[d:, _HALF:] = vt[:, 0:_CQ]

    qt = qt_ref[0, 0] * scale

    lm_sc = jax.lax.dot_general(
        lmk_ref[...], qt, (((0,), (0,)), ((), ())),
        preferred_element_type=jnp.float32,
    ) + lmm_ref[...]
    m2 = jnp.max(lm_sc, axis=0, keepdims=True)
    e2 = jnp.exp(lm_sc - m2)
    w2 = e2 / jnp.sum(e2, axis=0, keepdims=True)
    glob = jax.lax.dot_general(
        w2, lmv_ref[...], (((0,), (1,)), ((), ())),
        preferred_element_type=jnp.float32,
    )

    for c0 in range(nchunk):
        qc = qt[:, c0 * _CQ:(c0 + 1) * _CQ]
        if c0 == 0:
            kc = kv0_ref[0:d, :]
        else:
            kc = kt[:, c0 * _CQ - _HALF:c0 * _CQ + _CQ]
        sc_ref[:, c0 * _CQ:(c0 + 1) * _CQ] = jax.lax.dot_general(
            kc, qc, (((0,), (0,)), ((), ())),
            preferred_element_type=jnp.float32,
        )

    scm = sc_ref[...] + band_ref[...]
    m = jnp.max(scm, axis=0, keepdims=True)
    e = jnp.exp(scm - m)
    sc_ref[...] = e / jnp.sum(e, axis=0, keepdims=True)

    for c0 in range(nchunk):
        wc = sc_ref[:, c0 * _CQ:(c0 + 1) * _CQ]
        if c0 == 0:
            vc = kv0_ref[d:, :]
        else:
            vc = vt[:, c0 * _CQ - _HALF:c0 * _CQ + _CQ]
        loc = jax.lax.dot_general(
            wc, vc, (((0,), (1,)), ((), ())),
            preferred_element_type=jnp.float32,
        )
        o_ref[0, 0, c0 * _CQ:(c0 + 1) * _CQ, :] = (
            loc + glob[c0 * _CQ:(c0 + 1) * _CQ, :]
        ).astype(o_ref.dtype)


@jax.jit
def kernel(query, key, value):
    b, h, s, d = query.shape
    assert b == 1
    stride = s // _NUM_LANDMARKS

    qt = jnp.transpose(query, (0, 1, 3, 2))
    kt = jnp.transpose(key, (0, 1, 3, 2))
    vt = jnp.transpose(value, (0, 1, 3, 2))

    out = pl.pallas_call(
        functools.partial(_ssa_head_kernel, s=s, stride=stride),
        grid=(h,),
        in_specs=[
            pl.BlockSpec((1, 1, d, s), lambda hh: (0, hh, 0, 0)),
            pl.BlockSpec((1, 1, d, s), lambda hh: (0, hh, 0, 0)),
            pl.BlockSpec((1, 1, d, s), lambda hh: (0, hh, 0, 0)),
        ],
        out_specs=pl.BlockSpec((1, 1, s, d), lambda hh: (0, hh, 0, 0)),
        out_shape=jax.ShapeDtypeStruct((b, h, s, d), query.dtype),
        scratch_shapes=[
            pltpu.VMEM((_KW, s), jnp.float32),
            pltpu.VMEM((_NUM_LANDMARKS, s), jnp.float32),
            pltpu.VMEM((2 * d, _KW), jnp.float32),
            pltpu.VMEM((d, _NUM_LANDMARKS), jnp.float32),
            pltpu.VMEM((d, _NUM_LANDMARKS), jnp.float32),
            pltpu.VMEM((_KW, s), jnp.float32),
            pltpu.VMEM((s, _NUM_LANDMARKS), jnp.float32),
        ],
    )(qt, kt, vt)
    return out


# speedup vs baseline: 4.8532x; 1.7314x over previous
import functools
import math

import jax
import jax.numpy as jnp
from jax.experimental import pallas as pl
from jax.experimental.pallas import tpu as pltpu

_NUM_LANDMARKS = 64
_HALF = 32
_CQ = 128
_KW = _CQ + _HALF
_NEG = float("-inf")


def _ssa_head_kernel(qt_ref, kt_ref, vt_ref, o_ref,
                     band_ref, lmm_ref, kv0_ref, lmk_ref, lmv_ref, sc_ref,
                     sel_ref, *, s, stride):
    h = pl.program_id(0)
    d = qt_ref.shape[-2]
    scale = 1.0 / math.sqrt(d)
    nchunk = s // _CQ

    @pl.when(h == 0)
    def _():
        c = jax.lax.broadcasted_iota(jnp.int32, (_KW, s), 0)
        r = jax.lax.rem(jax.lax.broadcasted_iota(jnp.int32, (_KW, s), 1), _CQ)
        band_ref[...] = jnp.where((c >= r) & (c <= r + _HALF), 0.0, _NEG)
        ll = jax.lax.broadcasted_iota(jnp.int32, (_NUM_LANDMARKS, s), 0)
        rl = jax.lax.broadcasted_iota(jnp.int32, (_NUM_LANDMARKS, s), 1)
        lmm_ref[...] = jnp.where(ll * stride > rl, _NEG, 0.0)
        rs = jax.lax.broadcasted_iota(jnp.int32, (s, _NUM_LANDMARKS), 0)
        ls = jax.lax.broadcasted_iota(jnp.int32, (s, _NUM_LANDMARKS), 1)
        sel_ref[...] = (rs == ls * stride).astype(jnp.float32)

    kt = kt_ref[0, 0]
    vt = vt_ref[0, 0]
    lmk_ref[...] = jax.lax.dot_general(
        kt, sel_ref[...], (((1,), (0,)), ((), ())),
        preferred_element_type=jnp.float32)
    lmv_ref[...] = jax.lax.dot_general(
        vt, sel_ref[...], (((1,), (0,)), ((), ())),
        preferred_element_type=jnp.float32)
    zeros = jnp.zeros((d, _HALF), jnp.float32)
    kv0_ref[0:d, 0:_HALF] = zeros
    kv0_ref[d:, 0:_HALF] = zeros
    kv0_ref[0:d, _HALF:] = kt[:, 0:_CQ]
    kv0_ref[d:, _HALF:] = vt[:, 0:_CQ]

    qt = qt_ref[0, 0] * scale

    lm_sc = jax.lax.dot_general(
        lmk_ref[...], qt, (((0,), (0,)), ((), ())),
        preferred_element_type=jnp.float32,
    ) + lmm_ref[...]
    m2 = jnp.max(lm_sc, axis=0, keepdims=True)
    e2 = jnp.exp(lm_sc - m2)
    w2 = e2 / jnp.sum(e2, axis=0, keepdims=True)
    globt = jax.lax.dot_general(
        lmv_ref[...], w2, (((1,), (0,)), ((), ())),
        preferred_element_type=jnp.float32,
    )

    for c0 in range(nchunk):
        qc = qt[:, c0 * _CQ:(c0 + 1) * _CQ]
        if c0 == 0:
            kc = kv0_ref[0:d, :]
        else:
            kc = kt[:, c0 * _CQ - _HALF:c0 * _CQ + _CQ]
        sc_ref[:, c0 * _CQ:(c0 + 1) * _CQ] = jax.lax.dot_general(
            kc, qc, (((0,), (0,)), ((), ())),
            preferred_element_type=jnp.float32,
        )

    scm = sc_ref[...] + band_ref[...]
    m = jnp.max(scm, axis=0, keepdims=True)
    e = jnp.exp(scm - m)
    sc_ref[...] = e / jnp.sum(e, axis=0, keepdims=True)

    for c0 in range(nchunk):
        wc = sc_ref[:, c0 * _CQ:(c0 + 1) * _CQ]
        if c0 == 0:
            vc = kv0_ref[d:, :]
        else:
            vc = vt[:, c0 * _CQ - _HALF:c0 * _CQ + _CQ]
        loct = jax.lax.dot_general(
            vc, wc, (((1,), (0,)), ((), ())),
            preferred_element_type=jnp.float32,
        )
        o_ref[0, 0, :, c0 * _CQ:(c0 + 1) * _CQ] = (
            loct + globt[:, c0 * _CQ:(c0 + 1) * _CQ]
        ).astype(o_ref.dtype)


@jax.jit
def kernel(query, key, value):
    b, h, s, d = query.shape
    assert b == 1
    stride = s // _NUM_LANDMARKS

    qt = jnp.transpose(query, (0, 1, 3, 2))
    kt = jnp.transpose(key, (0, 1, 3, 2))
    vt = jnp.transpose(value, (0, 1, 3, 2))

    out = pl.pallas_call(
        functools.partial(_ssa_head_kernel, s=s, stride=stride),
        grid=(h,),
        in_specs=[
            pl.BlockSpec((1, 1, d, s), lambda hh: (0, hh, 0, 0)),
            pl.BlockSpec((1, 1, d, s), lambda hh: (0, hh, 0, 0)),
            pl.BlockSpec((1, 1, d, s), lambda hh: (0, hh, 0, 0)),
        ],
        out_specs=pl.BlockSpec((1, 1, d, s), lambda hh: (0, hh, 0, 0)),
        out_shape=jax.ShapeDtypeStruct((b, h, d, s), query.dtype),
        scratch_shapes=[
            pltpu.VMEM((_KW, s), jnp.float32),
            pltpu.VMEM((_NUM_LANDMARKS, s), jnp.float32),
            pltpu.VMEM((2 * d, _KW), jnp.float32),
            pltpu.VMEM((d, _NUM_LANDMARKS), jnp.float32),
            pltpu.VMEM((d, _NUM_LANDMARKS), jnp.float32),
            pltpu.VMEM((_KW, s), jnp.float32),
            pltpu.VMEM((s, _NUM_LANDMARKS), jnp.float32),
        ],
    )(qt, kt, vt)
    return jnp.transpose(out, (0, 1, 3, 2))
